# baseline (speedup 1.0000x reference)
import jax
import jax.numpy as jnp
from jax import lax
from jax.experimental import pallas as pl
from jax.experimental.pallas import tpu as pltpu
from jax.experimental.pallas import tpu_sc as plsc

BATCH = 4096
HIST = 200
LPAD = 208
NCH = LPAD // 16
EMBED = 64
TROW = 32
LANES = 16
NWORKERS = 32
ROWS_PER_W = BATCH // NWORKERS
NDC = EMBED // LANES

_BCAST_DNUMS = lax.GatherDimensionNumbers(
    offset_dims=(), collapsed_slice_dims=(0,), start_index_map=(0,))


def _bcast_lane(v, j):
    return lax.gather(v, jnp.full((LANES, 1), j, jnp.int32), _BCAST_DNUMS,
                      slice_sizes=(1,),
                      mode=lax.GatherScatterMode.PROMISE_IN_BOUNDS)


def _gelu(v):
    u = 0.7978845608028654 * (v + 0.044715 * v * v * v)
    e = jnp.exp(2.0 * u)
    t = 1.0 - 2.0 / (e + 1.0)
    return 0.5 * v * (1.0 + t)


def _sc_body(x_hbm, idx_hbm, tbl_hbm, out_hbm,
             x_v, idx_v, r0, r1, r2, r3, out_v, s0, s1, s2, s3):
    wid = lax.axis_index("s") * 2 + lax.axis_index("c")
    inbase = pl.multiple_of(wid * (ROWS_PER_W * LPAD), 128)
    obase = pl.multiple_of(wid * (ROWS_PER_W * EMBED), 128)
    pltpu.sync_copy(x_hbm.at[pl.ds(inbase, ROWS_PER_W * LPAD)], x_v)
    pltpu.sync_copy(idx_hbm.at[pl.ds(inbase, ROWS_PER_W * LPAD)], idx_v)

    def gather(row, dst, sem):
        start = pl.multiple_of(row * LPAD, 16)
        return pltpu.make_async_copy(tbl_hbm.at[idx_v.at[pl.ds(start, LPAD)]],
                                     dst, sem)

    slots = ((r0, s0), (r1, s1), (r2, s2), (r3, s3))
    for r in range(4):
        gather(r, *slots[r]).start()

    def accum_row(row, rows, accs):
        def chunk(k, accs):
            t0 = pl.multiple_of(row * LPAD + k * LANES, 16)
            w = x_v[pl.ds(t0, LANES)]
            iv = idx_v[pl.ds(t0, LANES)]
            w = jnp.where(iv != 0, w, 0.0)
            accs = list(accs)
            rbase = k * LANES
            for j in range(LANES):
                wb = _bcast_lane(w, j)
                r = rbase + j
                lo = plsc.bitcast(rows[r, pl.ds(0, 16)], jnp.bfloat16)
                hi = plsc.bitcast(rows[r, pl.ds(16, 16)], jnp.bfloat16)
                d0, d1 = plsc.unpack(lo, format=plsc.PackFormat.INTERLEAVED)
                d2, d3 = plsc.unpack(hi, format=plsc.PackFormat.INTERLEAVED)
                for dc, d in enumerate((d0, d1, d2, d3)):
                    accs[dc] = accs[dc] + wb * d
            return tuple(accs)
        return lax.fori_loop(0, NCH, chunk, tuple(accs))

    def finish_row(row, accs):
        for dc in range(NDC):
            o = pl.multiple_of(row * EMBED + dc * LANES, 16)
            out_v[pl.ds(o, LANES)] = _gelu(accs[dc])

    def zeros():
        return [jnp.zeros((LANES,), jnp.float32) for _ in range(NDC)]

    last = ROWS_PER_W - 1

    def body(i, carry):
        for k in range(4):
            row = 4 * i + k
            buf, sem = slots[k]
            gather(row, buf, sem).wait()
            accs = accum_row(row, buf, zeros())
            finish_row(row, accs)
            gather(jnp.minimum(row + 4, last), buf, sem).start()
        return carry

    lax.fori_loop(0, ROWS_PER_W // 4, body, 0)
    for k in range(4):
        gather(last, *slots[k]).wait()
    pltpu.sync_copy(out_v, out_hbm.at[pl.ds(obase, ROWS_PER_W * EMBED)])


def kernel(x, x_ind, table):
    xp = jnp.pad(x, ((0, 0), (0, LPAD - HIST))).reshape(-1)
    ip = jnp.pad(x_ind.astype(jnp.int32), ((0, 0), (0, LPAD - HIST))).reshape(-1)
    u = jax.lax.bitcast_convert_type(table, jnp.uint32)
    r = (u + jnp.uint32(0x7FFF) + ((u >> 16) & jnp.uint32(1))) >> 16
    t32 = jax.lax.bitcast_convert_type(r[:, 0::2] | (r[:, 1::2] << 16),
                                       jnp.int32)
    run = pl.kernel(
        _sc_body,
        out_type=jax.ShapeDtypeStruct((BATCH * EMBED,), jnp.float32),
        scratch_types=[
            pltpu.VMEM((ROWS_PER_W * LPAD,), jnp.float32),
            pltpu.VMEM((ROWS_PER_W * LPAD,), jnp.int32),
            pltpu.VMEM((LPAD, TROW), jnp.int32),
            pltpu.VMEM((LPAD, TROW), jnp.int32),
            pltpu.VMEM((LPAD, TROW), jnp.int32),
            pltpu.VMEM((LPAD, TROW), jnp.int32),
            pltpu.VMEM((ROWS_PER_W * EMBED,), jnp.float32),
            pltpu.SemaphoreType.DMA,
            pltpu.SemaphoreType.DMA,
            pltpu.SemaphoreType.DMA,
            pltpu.SemaphoreType.DMA,
        ],
        mesh=plsc.VectorSubcoreMesh(core_axis_name="c", subcore_axis_name="s"),
        compiler_params=pltpu.CompilerParams(use_tc_tiling_on_sc=False,
                                             needs_layout_passes=False),
    )
    out = run(xp, ip, t32).reshape(BATCH, 2, 2, LANES)
    return out.transpose(0, 1, 3, 2).reshape(BATCH, EMBED)


# speedup vs baseline: 5.7603x; 5.7603x over previous
import jax
import jax.numpy as jnp
from jax import lax
from jax.experimental import pallas as pl
from jax.experimental.pallas import tpu as pltpu
from jax.experimental.pallas import tpu_sc as plsc

BATCH = 4096
HIST = 200
LPAD = 208
NCH = LPAD // 16
EMBED = 64
TROW = 32
LANES = 16
NWORKERS = 32
ROWS_PER_W = BATCH // NWORKERS
NDC = EMBED // LANES

_BCAST_DNUMS = lax.GatherDimensionNumbers(
    offset_dims=(), collapsed_slice_dims=(0,), start_index_map=(0,))


def _bcast_lane(v, j):
    return lax.gather(v, jnp.full((LANES, 1), j, jnp.int32), _BCAST_DNUMS,
                      slice_sizes=(1,),
                      mode=lax.GatherScatterMode.PROMISE_IN_BOUNDS)


def _gelu(v):
    u = 0.7978845608028654 * (v + 0.044715 * v * v * v)
    e = jnp.exp(2.0 * u)
    t = 1.0 - 2.0 / (e + 1.0)
    return 0.5 * v * (1.0 + t)


def _sc_body(x_hbm, idx_hbm, tbl_hbm, out_hbm,
             x_v, idx_v, r0, r1, r2, r3, out_v, s0, s1, s2, s3):
    wid = lax.axis_index("s") * 2 + lax.axis_index("c")
    inbase = pl.multiple_of(wid * (ROWS_PER_W * LPAD), 128)
    obase = pl.multiple_of(wid * (ROWS_PER_W * EMBED), 128)
    pltpu.sync_copy(x_hbm.at[pl.ds(inbase, ROWS_PER_W * LPAD)], x_v)
    pltpu.sync_copy(idx_hbm.at[pl.ds(inbase, ROWS_PER_W * LPAD)], idx_v)

    def gather(row, dst, sem):
        start = pl.multiple_of(row * LPAD, 16)
        return pltpu.make_async_copy(tbl_hbm.at[idx_v.at[pl.ds(start, LPAD)]],
                                     dst, sem)

    slots = ((r0, s0), (r1, s1), (r2, s2), (r3, s3))
    for r in range(4):
        gather(r, *slots[r]).start()

    def accum_row(row, rows, accs):
        def chunk(k, accs):
            t0 = pl.multiple_of(row * LPAD + k * LANES, 16)
            w = x_v[pl.ds(t0, LANES)]
            iv = idx_v[pl.ds(t0, LANES)]
            w = jnp.where(iv != 0, w, 0.0)
            accs = list(accs)
            rbase = k * LANES
            for j in range(LANES):
                wb = _bcast_lane(w, j)
                r = rbase + j
                lo = plsc.bitcast(rows[r, pl.ds(0, 16)], jnp.bfloat16)
                hi = plsc.bitcast(rows[r, pl.ds(16, 16)], jnp.bfloat16)
                d0, d2 = plsc.unpack(lo, format=plsc.PackFormat.INTERLEAVED)
                d1, d3 = plsc.unpack(hi, format=plsc.PackFormat.INTERLEAVED)
                for dc, d in enumerate((d0, d1, d2, d3)):
                    accs[dc] = accs[dc] + wb * d
            return tuple(accs)
        return lax.fori_loop(0, NCH, chunk, tuple(accs))

    def finish_row(row, accs):
        for dc in range(NDC):
            o = pl.multiple_of(row * EMBED + dc * LANES, 16)
            out_v[pl.ds(o, LANES)] = _gelu(accs[dc])

    def zeros():
        return [jnp.zeros((LANES,), jnp.float32) for _ in range(NDC)]

    last = ROWS_PER_W - 1

    def body(i, carry):
        for k in range(4):
            row = 4 * i + k
            buf, sem = slots[k]
            gather(row, buf, sem).wait()
            accs = accum_row(row, buf, zeros())
            finish_row(row, accs)
            gather(jnp.minimum(row + 4, last), buf, sem).start()
        return carry

    lax.fori_loop(0, ROWS_PER_W // 4, body, 0)
    for k in range(4):
        gather(last, *slots[k]).wait()
    pltpu.sync_copy(out_v, out_hbm.at[pl.ds(obase, ROWS_PER_W * EMBED)])


def kernel(x, x_ind, table):
    xp = jnp.pad(x, ((0, 0), (0, LPAD - HIST))).reshape(-1)
    ip = jnp.pad(x_ind.astype(jnp.int32), ((0, 0), (0, LPAD - HIST))).reshape(-1)
    u = jax.lax.bitcast_convert_type(table, jnp.uint32)
    r = (u + jnp.uint32(0x7FFF) + ((u >> 16) & jnp.uint32(1))) >> 16
    t32 = jax.lax.bitcast_convert_type(
        r[:, :TROW] | (r[:, TROW:] << 16), jnp.int32)
    run = pl.kernel(
        _sc_body,
        out_type=jax.ShapeDtypeStruct((BATCH * EMBED,), jnp.float32),
        scratch_types=[
            pltpu.VMEM((ROWS_PER_W * LPAD,), jnp.float32),
            pltpu.VMEM((ROWS_PER_W * LPAD,), jnp.int32),
            pltpu.VMEM((LPAD, TROW), jnp.int32),
            pltpu.VMEM((LPAD, TROW), jnp.int32),
            pltpu.VMEM((LPAD, TROW), jnp.int32),
            pltpu.VMEM((LPAD, TROW), jnp.int32),
            pltpu.VMEM((ROWS_PER_W * EMBED,), jnp.float32),
            pltpu.SemaphoreType.DMA,
            pltpu.SemaphoreType.DMA,
            pltpu.SemaphoreType.DMA,
            pltpu.SemaphoreType.DMA,
        ],
        mesh=plsc.VectorSubcoreMesh(core_axis_name="c", subcore_axis_name="s"),
        compiler_params=pltpu.CompilerParams(use_tc_tiling_on_sc=False,
                                             needs_layout_passes=False),
    )
    return run(xp, ip, t32).reshape(BATCH, EMBED)


# speedup vs baseline: 8.7404x; 1.5174x over previous
import jax
import jax.numpy as jnp
from jax import lax
from jax.experimental import pallas as pl
from jax.experimental.pallas import tpu as pltpu
from jax.experimental.pallas import tpu_sc as plsc

BATCH = 4096
HIST = 200
LPAD = 208
NCH = LPAD // 16
EMBED = 64
TROW = 32
LANES = 16
NWORKERS = 32
ROWS_PER_W = BATCH // NWORKERS
NDC = EMBED // LANES

_BCAST_DNUMS = lax.GatherDimensionNumbers(
    offset_dims=(), collapsed_slice_dims=(0,), start_index_map=(0,))


def _bcast_lane(v, j):
    return lax.gather(v, jnp.full((LANES, 1), j, jnp.int32), _BCAST_DNUMS,
                      slice_sizes=(1,),
                      mode=lax.GatherScatterMode.PROMISE_IN_BOUNDS)


def _gelu(v):
    u = 0.7978845608028654 * (v + 0.044715 * v * v * v)
    e = jnp.exp(2.0 * u)
    t = 1.0 - 2.0 / (e + 1.0)
    return 0.5 * v * (1.0 + t)


def _sc_body(x_hbm, idx_hbm, tbl_hbm, out_hbm,
             x_v, idx_v, r0, r1, r2, r3, out_v, s0, s1, s2, s3):
    wid = lax.axis_index("s") * 2 + lax.axis_index("c")
    inbase = pl.multiple_of(wid * (ROWS_PER_W * LPAD), 128)
    obase = pl.multiple_of(wid * (ROWS_PER_W * EMBED), 128)
    pltpu.sync_copy(x_hbm.at[pl.ds(inbase, ROWS_PER_W * LPAD)], x_v)
    pltpu.sync_copy(idx_hbm.at[pl.ds(inbase, ROWS_PER_W * LPAD)], idx_v)

    def gather(row, dst, sem):
        start = pl.multiple_of(row * LPAD, 16)
        return pltpu.make_async_copy(tbl_hbm.at[idx_v.at[pl.ds(start, LPAD)]],
                                     dst, sem)

    slots = ((r0, s0), (r1, s1), (r2, s2), (r3, s3))
    for r in range(4):
        gather(r, *slots[r]).start()

    def accum_row(row, rows, accs):
        def chunk(k, accs):
            t0 = pl.multiple_of(row * LPAD + k * LANES, 16)
            w = x_v[pl.ds(t0, LANES)]
            iv = idx_v[pl.ds(t0, LANES)]
            w = jnp.where(iv != 0, w, 0.0)
            accs = list(accs)
            rbase = k * LANES
            for j in range(LANES):
                wb = _bcast_lane(w, j)
                r = rbase + j
                lo = plsc.bitcast(rows[r, pl.ds(0, 16)], jnp.bfloat16)
                hi = plsc.bitcast(rows[r, pl.ds(16, 16)], jnp.bfloat16)
                d0, d1 = plsc.unpack(lo, format=plsc.PackFormat.INTERLEAVED)
                d2, d3 = plsc.unpack(hi, format=plsc.PackFormat.INTERLEAVED)
                for dc, d in enumerate((d0, d1, d2, d3)):
                    accs[dc] = accs[dc] + wb * d
            return tuple(accs)
        return lax.fori_loop(0, NCH, chunk, tuple(accs))

    def finish_row(row, accs):
        for dc in range(NDC):
            o = pl.multiple_of(row * EMBED + dc * LANES, 16)
            out_v[pl.ds(o, LANES)] = _gelu(accs[dc])

    def zeros():
        return [jnp.zeros((LANES,), jnp.float32) for _ in range(NDC)]

    last = ROWS_PER_W - 1

    def body(i, carry):
        for k in range(4):
            row = 4 * i + k
            buf, sem = slots[k]
            gather(row, buf, sem).wait()
            accs = accum_row(row, buf, zeros())
            finish_row(row, accs)
            gather(jnp.minimum(row + 4, last), buf, sem).start()
        return carry

    lax.fori_loop(0, ROWS_PER_W // 4, body, 0)
    for k in range(4):
        gather(last, *slots[k]).wait()
    pltpu.sync_copy(out_v, out_hbm.at[pl.ds(obase, ROWS_PER_W * EMBED)])


def kernel(x, x_ind, table):
    xp = jnp.pad(x, ((0, 0), (0, LPAD - HIST))).reshape(-1)
    ip = jnp.pad(x_ind.astype(jnp.int32), ((0, 0), (0, LPAD - HIST))).reshape(-1)
    v = table.shape[0]
    tb = table.astype(jnp.bfloat16)
    tb = tb.reshape(v, 2, 2, LANES).transpose(0, 1, 3, 2).reshape(v, EMBED)
    t32 = jax.lax.bitcast_convert_type(tb.reshape(v, TROW, 2), jnp.int32)
    run = pl.kernel(
        _sc_body,
        out_type=jax.ShapeDtypeStruct((BATCH * EMBED,), jnp.float32),
        scratch_types=[
            pltpu.VMEM((ROWS_PER_W * LPAD,), jnp.float32),
            pltpu.VMEM((ROWS_PER_W * LPAD,), jnp.int32),
            pltpu.VMEM((LPAD, TROW), jnp.int32),
            pltpu.VMEM((LPAD, TROW), jnp.int32),
            pltpu.VMEM((LPAD, TROW), jnp.int32),
            pltpu.VMEM((LPAD, TROW), jnp.int32),
            pltpu.VMEM((ROWS_PER_W * EMBED,), jnp.float32),
            pltpu.SemaphoreType.DMA,
            pltpu.SemaphoreType.DMA,
            pltpu.SemaphoreType.DMA,
            pltpu.SemaphoreType.DMA,
        ],
        mesh=plsc.VectorSubcoreMesh(core_axis_name="c", subcore_axis_name="s"),
        compiler_params=pltpu.CompilerParams(use_tc_tiling_on_sc=False,
                                             needs_layout_passes=False),
    )
    return run(xp, ip, t32).reshape(BATCH, EMBED)
